# Initial kernel scaffold; baseline (speedup 1.0000x reference)
import jax
import jax.numpy as jnp
from jax.experimental import pallas as pl


def kernel(x, edge_index, enc_W1, enc_b1, enc_W2, enc_b2, enc_W3, enc_b3, lat_W1, lat_b1, lat_W2, lat_b2, dfc_W1, dfc_b1, dfc_W2, dfc_b2, dec_W1, dec_b1, dec_W2, dec_b2, dec_W3, dec_b3):
    raise NotImplementedError("write your pallas kernel here")


# speedup vs baseline: 10.8133x; 10.8133x over previous
import functools

import jax
import jax.numpy as jnp
from jax import lax
from jax.experimental import pallas as pl
from jax.experimental.pallas import tpu as pltpu
from jax.experimental.pallas import tpu_sc as plsc

NN = 10000
EE = 320000
CH = 128
NCHUNK = EE // CH
PER_CORE = NCHUNK // 2
BASE_CH = PER_CORE // 16
REM_CH = PER_CORE - BASE_CH * 16
RPT = 624
ZR = 48
BLK = 1000


def _mesh():
    return plsc.VectorSubcoreMesh(core_axis_name="c", subcore_axis_name="s")


@functools.lru_cache(maxsize=None)
def _make_prop(dim):
    @functools.partial(
        pl.kernel,
        mesh=_mesh(),
        out_type=jax.ShapeDtypeStruct((2, NN, dim), jnp.float32),
        scratch_types=[
            pltpu.VMEM((CH,), jnp.int32),
            pltpu.VMEM((CH,), jnp.int32),
            pltpu.VMEM((CH, dim), jnp.float32),
            pltpu.VMEM((ZR, dim), jnp.float32),
            pltpu.VMEM_SHARED((NN, dim), jnp.float32),
            pltpu.SemaphoreType.DMA,
        ],
    )
    def prop(h_hbm, src_hbm, dst_hbm, out_hbm, src_v, dst_v, rows_v, zb, acc, sem):
        cid = lax.axis_index("c")
        sid = lax.axis_index("s")
        zv = jnp.zeros((16,), jnp.float32)
        for r in range(ZR):
            for k in range(dim // 16):
                zb[r, pl.ds(k * 16, 16)] = zv
        row0 = sid * RPT
        for r in range(RPT // ZR):
            pltpu.sync_copy(zb, acc.at[pl.ds(row0 + r * ZR, ZR)])

        @pl.when(sid == 0)
        def _():
            pltpu.sync_copy(zb.at[pl.ds(0, 16)], acc.at[pl.ds(16 * RPT, 16)])

        plsc.subcore_barrier()

        start = cid * PER_CORE + sid * BASE_CH + jnp.minimum(sid, REM_CH)

        def chunk(j):
            pltpu.sync_copy(src_hbm.at[j], src_v)
            pltpu.sync_copy(dst_hbm.at[j], dst_v)
            pltpu.async_copy(h_hbm.at[src_v], rows_v, sem).wait()
            pltpu.sync_copy(rows_v, acc.at[dst_v], add=True)

        def body(i, carry):
            chunk(start + i)
            return carry

        lax.fori_loop(0, BASE_CH, body, 0)

        @pl.when(sid < REM_CH)
        def _():
            chunk(start + BASE_CH)

        plsc.subcore_barrier()
        pltpu.sync_copy(acc.at[pl.ds(row0, RPT)],
                        out_hbm.at[cid, pl.ds(row0, RPT)])

        @pl.when(sid == 0)
        def _():
            pltpu.sync_copy(acc.at[pl.ds(16 * RPT, 16)],
                            out_hbm.at[cid, pl.ds(16 * RPT, 16)])

    return prop


@functools.lru_cache(maxsize=None)
def _make_deg():
    @functools.partial(
        pl.kernel,
        mesh=_mesh(),
        out_type=jax.ShapeDtypeStruct((2, NN, 128), jnp.float32),
        scratch_types=[
            pltpu.VMEM((CH,), jnp.int32),
            pltpu.VMEM((CH, 128), jnp.float32),
            pltpu.VMEM((ZR, 128), jnp.float32),
            pltpu.VMEM_SHARED((NN, 128), jnp.float32),
        ],
    )
    def deg(dst_hbm, out_hbm, dst_v, ones_v, zb, acc):
        cid = lax.axis_index("c")
        sid = lax.axis_index("s")
        zv = jnp.zeros((16,), jnp.float32)
        ov = jnp.ones((16,), jnp.float32)
        for r in range(ZR):
            for k in range(8):
                zb[r, pl.ds(k * 16, 16)] = zv
        for r in range(CH):
            for k in range(8):
                ones_v[r, pl.ds(k * 16, 16)] = ov
        row0 = sid * RPT
        for r in range(RPT // ZR):
            pltpu.sync_copy(zb, acc.at[pl.ds(row0 + r * ZR, ZR)])

        @pl.when(sid == 0)
        def _():
            pltpu.sync_copy(zb.at[pl.ds(0, 16)], acc.at[pl.ds(16 * RPT, 16)])

        plsc.subcore_barrier()

        start = cid * PER_CORE + sid * BASE_CH + jnp.minimum(sid, REM_CH)

        def chunk(j):
            pltpu.sync_copy(dst_hbm.at[j], dst_v)
            pltpu.sync_copy(ones_v, acc.at[dst_v], add=True)

        def body(i, carry):
            chunk(start + i)
            return carry

        lax.fori_loop(0, BASE_CH, body, 0)

        @pl.when(sid < REM_CH)
        def _():
            chunk(start + BASE_CH)

        plsc.subcore_barrier()
        pltpu.sync_copy(acc.at[pl.ds(row0, RPT)],
                        out_hbm.at[cid, pl.ds(row0, RPT)])

        @pl.when(sid == 0)
        def _():
            pltpu.sync_copy(acc.at[pl.ds(16 * RPT, 16)],
                            out_hbm.at[cid, pl.ds(16 * RPT, 16)])

    return deg


def _row_spec(d):
    return pl.BlockSpec((BLK, d), lambda i: (i, 0))


def _full_spec(shape):
    nd = len(shape)
    return pl.BlockSpec(shape, lambda i, _nd=nd: (0,) * _nd)


def _lrelu(v):
    return jnp.where(v >= 0, v, 0.01 * v)


def _padc(w):
    if w.ndim == 1:
        return jnp.pad(w, (0, 128 - w.shape[0]))
    return jnp.pad(w, ((0, 0), (0, 128 - w.shape[1])))


def _padr(w):
    return jnp.pad(w, ((0, 128 - w.shape[0]), (0, 0)))


def _tc_pre(d0, d1, x, w1):
    def body(d0_ref, d1_ref, x_ref, w_ref, hp_ref, dinv_ref):
        deg = d0_ref[:, 0:1] + d1_ref[:, 0:1] + 1.0
        dinv = 1.0 / jnp.sqrt(deg)
        h = jnp.dot(x_ref[...], w_ref[...], preferred_element_type=jnp.float32)
        hp_ref[...] = dinv * h
        dinv_ref[...] = dinv

    return pl.pallas_call(
        body,
        grid=(NN // BLK,),
        in_specs=[_row_spec(128), _row_spec(128), _row_spec(128),
                  _full_spec((128, 128))],
        out_specs=[_row_spec(128), _row_spec(1)],
        out_shape=[jax.ShapeDtypeStruct((NN, 128), jnp.float32),
                   jax.ShapeDtypeStruct((NN, 1), jnp.float32)],
    )(d0, d1, x, w1)


def _tc_layer(p0, p1, hp, dinv, b, w):
    def body(p0_ref, p1_ref, hp_ref, dinv_ref, b_ref, w_ref, o_ref):
        dv = dinv_ref[...]
        z = dv * (p0_ref[...] + p1_ref[...] + hp_ref[...]) + b_ref[...]
        a = _lrelu(z)
        o_ref[...] = dv * jnp.dot(a, w_ref[...],
                                  preferred_element_type=jnp.float32)

    return pl.pallas_call(
        body,
        grid=(NN // BLK,),
        in_specs=[_row_spec(128), _row_spec(128), _row_spec(128),
                  _row_spec(1), _full_spec((1, 128)), _full_spec((128, 128))],
        out_specs=_row_spec(128),
        out_shape=jax.ShapeDtypeStruct((NN, 128), jnp.float32),
    )(p0, p1, hp, dinv, b.reshape(1, 128), w)


def _tc_mid(p0, p1, hp, dinv, b3, lw1, lb1, lw2, lb2, fw1, fb1, fw2, fb2, dw1):
    def body(p0_ref, p1_ref, hp_ref, dinv_ref, b3_ref,
             lw1_ref, lb1_ref, lw2_ref, lb2_ref,
             fw1_ref, fb1_ref, fw2_ref, fb2_ref, dw1_ref, o_ref):
        dv = dinv_ref[...]
        z = dv * (p0_ref[...] + p1_ref[...] + hp_ref[...]) + b3_ref[...]
        t = _lrelu(z)
        t = _lrelu(jnp.dot(t, lw1_ref[...],
                           preferred_element_type=jnp.float32) + lb1_ref[...])
        t = jnp.dot(t, lw2_ref[...],
                    preferred_element_type=jnp.float32) + lb2_ref[...]
        t = _lrelu(jnp.dot(t, fw1_ref[...],
                           preferred_element_type=jnp.float32) + fb1_ref[...])
        t = _lrelu(jnp.dot(t, fw2_ref[...],
                           preferred_element_type=jnp.float32) + fb2_ref[...])
        o_ref[...] = dv * jnp.dot(t, dw1_ref[...],
                                  preferred_element_type=jnp.float32)

    return pl.pallas_call(
        body,
        grid=(NN // BLK,),
        in_specs=[_row_spec(128), _row_spec(128), _row_spec(128), _row_spec(1),
                  _full_spec((1, 128)),
                  _full_spec((128, 32)), _full_spec((1, 32)),
                  _full_spec((32, 3)), _full_spec((1, 3)),
                  _full_spec((3, 32)), _full_spec((1, 32)),
                  _full_spec((32, 64)), _full_spec((1, 64)),
                  _full_spec((64, 128))],
        out_specs=_row_spec(128),
        out_shape=jax.ShapeDtypeStruct((NN, 128), jnp.float32),
    )(p0, p1, hp, dinv, b3.reshape(1, 128),
      lw1, lb1.reshape(1, 32), lw2, lb2.reshape(1, 3),
      fw1, fb1.reshape(1, 32), fw2, fb2.reshape(1, 64), dw1)


def _tc_final(p0, p1, hp, dinv, b):
    def body(p0_ref, p1_ref, hp_ref, dinv_ref, b_ref, o_ref):
        dv = dinv_ref[...]
        z = dv * (p0_ref[...] + p1_ref[...] + hp_ref[...]) + b_ref[...]
        o_ref[...] = jnp.tanh(_lrelu(z))

    return pl.pallas_call(
        body,
        grid=(NN // BLK,),
        in_specs=[_row_spec(128), _row_spec(128), _row_spec(128),
                  _row_spec(1), _full_spec((1, 128))],
        out_specs=_row_spec(128),
        out_shape=jax.ShapeDtypeStruct((NN, 128), jnp.float32),
    )(p0, p1, hp, dinv, b.reshape(1, 128))


def kernel(x, edge_index, enc_W1, enc_b1, enc_W2, enc_b2, enc_W3, enc_b3,
           lat_W1, lat_b1, lat_W2, lat_b2, dfc_W1, dfc_b1, dfc_W2, dfc_b2,
           dec_W1, dec_b1, dec_W2, dec_b2, dec_W3, dec_b3):
    src = edge_index[0].reshape(NCHUNK, CH)
    dst = edge_index[1].reshape(NCHUNK, CH)

    prop = _make_prop(128)

    degp = _make_deg()(dst)
    hp1, dinv = _tc_pre(degp[0], degp[1], x, _padc(enc_W1))

    agg = prop(hp1, src, dst)
    hp2 = _tc_layer(agg[0], agg[1], hp1, dinv, _padc(enc_b1), _padr(enc_W2))

    agg = prop(hp2, src, dst)
    hp3 = _tc_layer(agg[0], agg[1], hp2, dinv, enc_b2, _padc(enc_W3))

    agg = prop(hp3, src, dst)
    hp4 = _tc_mid(agg[0], agg[1], hp3, dinv, _padc(enc_b3),
                  _padr(lat_W1), lat_b1, lat_W2, lat_b2,
                  dfc_W1, dfc_b1, dfc_W2, dfc_b2, dec_W1)

    agg = prop(hp4, src, dst)
    hp5 = _tc_layer(agg[0], agg[1], hp4, dinv, dec_b1, _padc(dec_W2))

    agg = prop(hp5, src, dst)
    hp6 = _tc_layer(agg[0], agg[1], hp5, dinv, _padc(dec_b2), _padr(dec_W3))

    agg = prop(hp6, src, dst)
    return _tc_final(agg[0], agg[1], hp6, dinv, dec_b3)


# speedup vs baseline: 16.6168x; 1.5367x over previous
import functools

import jax
import jax.numpy as jnp
from jax import lax
from jax.experimental import pallas as pl
from jax.experimental.pallas import tpu as pltpu
from jax.experimental.pallas import tpu_sc as plsc

NN = 10000
EE = 320000
CH = 128
NCHUNK = EE // CH
BASE_CH = NCHUNK // 32
PER_CORE = BASE_CH * 16
LEFT0 = 2 * PER_CORE
RPT = 624
ZR = 48
BLK = 1000


def _mesh():
    return plsc.VectorSubcoreMesh(core_axis_name="c", subcore_axis_name="s")


@functools.lru_cache(maxsize=None)
def _make_prop(dim):
    @functools.partial(
        pl.kernel,
        mesh=_mesh(),
        out_type=jax.ShapeDtypeStruct((2, NN, dim), jnp.float32),
        scratch_types=[
            pltpu.VMEM((CH,), jnp.int32),
            pltpu.VMEM((CH,), jnp.int32),
            pltpu.VMEM((CH, dim), jnp.float32),
            pltpu.VMEM((CH,), jnp.int32),
            pltpu.VMEM((CH,), jnp.int32),
            pltpu.VMEM((CH, dim), jnp.float32),
            pltpu.VMEM((ZR, dim), jnp.float32),
            pltpu.VMEM_SHARED((NN, dim), jnp.float32),
            pltpu.SemaphoreType.DMA,
            pltpu.SemaphoreType.DMA,
            pltpu.SemaphoreType.DMA,
            pltpu.SemaphoreType.DMA,
        ],
    )
    def prop(h_hbm, src_hbm, dst_hbm, out_hbm,
             sa, da, ra, sb, db, rb, zb, acc, gsa, gsb, ssa, ssb):
        cid = lax.axis_index("c")
        sid = lax.axis_index("s")
        zv = jnp.zeros((16,), jnp.float32)
        for r in range(ZR):
            for k in range(dim // 16):
                zb[r, pl.ds(k * 16, 16)] = zv
        row0 = sid * RPT
        for r in range(RPT // ZR):
            pltpu.sync_copy(zb, acc.at[pl.ds(row0 + r * ZR, ZR)])

        @pl.when(sid == 0)
        def _():
            pltpu.sync_copy(zb.at[pl.ds(0, 16)], acc.at[pl.ds(16 * RPT, 16)])

        plsc.subcore_barrier()

        start = (cid * 16 + sid) * BASE_CH
        A = (sa, da, ra, gsa, ssa)
        B = (sb, db, rb, gsb, ssb)

        def idx_load(j, S):
            pltpu.sync_copy(src_hbm.at[j], S[0])
            pltpu.sync_copy(dst_hbm.at[j], S[1])

        def gather_start(S):
            pltpu.async_copy(h_hbm.at[S[0]], S[2], S[3])

        def gather_wait(S):
            pltpu.make_async_copy(h_hbm.at[S[0]], S[2], S[3]).wait()

        def scat_start(S):
            pltpu.async_copy(S[2], acc.at[S[1]], S[4], add=True)

        def scat_wait(S):
            pltpu.make_async_copy(S[2], acc.at[S[1]], S[4]).wait()

        def stage(k, X, Y, first=False):
            if not first:
                scat_wait(Y)
            idx_load(k + 1, Y)
            gather_wait(X)
            gather_start(Y)
            scat_start(X)

        idx_load(start, A)
        gather_start(A)
        stage(start, A, B, first=True)

        def body(m, carry):
            t = start + 1 + 2 * m
            stage(t, B, A)
            stage(t + 1, A, B)
            return carry

        lax.fori_loop(0, (BASE_CH - 2) // 2, body, 0)

        gather_wait(B)
        scat_wait(A)
        scat_start(B)
        scat_wait(B)

        @pl.when((cid == 0) & (sid < NCHUNK - 32 * BASE_CH))
        def _():
            idx_load(LEFT0 + sid, A)
            pltpu.async_copy(h_hbm.at[sa], ra, gsa).wait()
            pltpu.sync_copy(ra, acc.at[da], add=True)

        plsc.subcore_barrier()
        pltpu.sync_copy(acc.at[pl.ds(row0, RPT)],
                        out_hbm.at[cid, pl.ds(row0, RPT)])

        @pl.when(sid == 0)
        def _():
            pltpu.sync_copy(acc.at[pl.ds(16 * RPT, 16)],
                            out_hbm.at[cid, pl.ds(16 * RPT, 16)])

    return prop


@functools.lru_cache(maxsize=None)
def _make_deg():
    @functools.partial(
        pl.kernel,
        mesh=_mesh(),
        out_type=jax.ShapeDtypeStruct((2, NN, 128), jnp.float32),
        scratch_types=[
            pltpu.VMEM((CH,), jnp.int32),
            pltpu.VMEM((CH,), jnp.int32),
            pltpu.VMEM((CH, 128), jnp.float32),
            pltpu.VMEM((ZR, 128), jnp.float32),
            pltpu.VMEM_SHARED((NN, 128), jnp.float32),
            pltpu.SemaphoreType.DMA,
            pltpu.SemaphoreType.DMA,
        ],
    )
    def deg(dst_hbm, out_hbm, da, db, ones_v, zb, acc, ssa, ssb):
        cid = lax.axis_index("c")
        sid = lax.axis_index("s")
        zv = jnp.zeros((16,), jnp.float32)
        ov = jnp.ones((16,), jnp.float32)
        for r in range(ZR):
            for k in range(8):
                zb[r, pl.ds(k * 16, 16)] = zv
        for r in range(CH):
            for k in range(8):
                ones_v[r, pl.ds(k * 16, 16)] = ov
        row0 = sid * RPT
        for r in range(RPT // ZR):
            pltpu.sync_copy(zb, acc.at[pl.ds(row0 + r * ZR, ZR)])

        @pl.when(sid == 0)
        def _():
            pltpu.sync_copy(zb.at[pl.ds(0, 16)], acc.at[pl.ds(16 * RPT, 16)])

        plsc.subcore_barrier()

        start = (cid * 16 + sid) * BASE_CH

        def scat_start(d, sem):
            pltpu.async_copy(ones_v, acc.at[d], sem, add=True)

        def scat_wait(d, sem):
            pltpu.make_async_copy(ones_v, acc.at[d], sem).wait()

        pltpu.sync_copy(dst_hbm.at[start], da)
        scat_start(da, ssa)
        pltpu.sync_copy(dst_hbm.at[start + 1], db)
        scat_start(db, ssb)

        def body(m, carry):
            t = start + 2 + 2 * m
            scat_wait(da, ssa)
            pltpu.sync_copy(dst_hbm.at[t], da)
            scat_start(da, ssa)
            scat_wait(db, ssb)
            pltpu.sync_copy(dst_hbm.at[t + 1], db)
            scat_start(db, ssb)
            return carry

        lax.fori_loop(0, (BASE_CH - 2) // 2, body, 0)
        scat_wait(da, ssa)
        scat_wait(db, ssb)

        @pl.when((cid == 0) & (sid < NCHUNK - 32 * BASE_CH))
        def _():
            pltpu.sync_copy(dst_hbm.at[LEFT0 + sid], da)
            pltpu.sync_copy(ones_v, acc.at[da], add=True)

        plsc.subcore_barrier()
        pltpu.sync_copy(acc.at[pl.ds(row0, RPT)],
                        out_hbm.at[cid, pl.ds(row0, RPT)])

        @pl.when(sid == 0)
        def _():
            pltpu.sync_copy(acc.at[pl.ds(16 * RPT, 16)],
                            out_hbm.at[cid, pl.ds(16 * RPT, 16)])

    return deg


def _row_spec(d):
    return pl.BlockSpec((BLK, d), lambda i: (i, 0))


def _full_spec(shape):
    nd = len(shape)
    return pl.BlockSpec(shape, lambda i, _nd=nd: (0,) * _nd)


def _lrelu(v):
    return jnp.where(v >= 0, v, 0.01 * v)


def _padc(w):
    if w.ndim == 1:
        return jnp.pad(w, (0, 128 - w.shape[0]))
    return jnp.pad(w, ((0, 0), (0, 128 - w.shape[1])))


def _padr(w):
    return jnp.pad(w, ((0, 128 - w.shape[0]), (0, 0)))


def _tc_pre(d0, d1, x, w1):
    def body(d0_ref, d1_ref, x_ref, w_ref, hp_ref, dinv_ref):
        deg = d0_ref[:, 0:1] + d1_ref[:, 0:1] + 1.0
        dinv = 1.0 / jnp.sqrt(deg)
        h = jnp.dot(x_ref[...], w_ref[...], preferred_element_type=jnp.float32)
        hp_ref[...] = dinv * h
        dinv_ref[...] = dinv

    return pl.pallas_call(
        body,
        grid=(NN // BLK,),
        in_specs=[_row_spec(128), _row_spec(128), _row_spec(128),
                  _full_spec((128, 128))],
        out_specs=[_row_spec(128), _row_spec(1)],
        out_shape=[jax.ShapeDtypeStruct((NN, 128), jnp.float32),
                   jax.ShapeDtypeStruct((NN, 1), jnp.float32)],
    )(d0, d1, x, w1)


def _tc_layer(p0, p1, hp, dinv, b, w):
    def body(p0_ref, p1_ref, hp_ref, dinv_ref, b_ref, w_ref, o_ref):
        dv = dinv_ref[...]
        z = dv * (p0_ref[...] + p1_ref[...] + hp_ref[...]) + b_ref[...]
        a = _lrelu(z)
        o_ref[...] = dv * jnp.dot(a, w_ref[...],
                                  preferred_element_type=jnp.float32)

    return pl.pallas_call(
        body,
        grid=(NN // BLK,),
        in_specs=[_row_spec(128), _row_spec(128), _row_spec(128),
                  _row_spec(1), _full_spec((1, 128)), _full_spec((128, 128))],
        out_specs=_row_spec(128),
        out_shape=jax.ShapeDtypeStruct((NN, 128), jnp.float32),
    )(p0, p1, hp, dinv, b.reshape(1, 128), w)


def _tc_mid(p0, p1, hp, dinv, b3, lw1, lb1, lw2, lb2, fw1, fb1, fw2, fb2, dw1):
    def body(p0_ref, p1_ref, hp_ref, dinv_ref, b3_ref,
             lw1_ref, lb1_ref, lw2_ref, lb2_ref,
             fw1_ref, fb1_ref, fw2_ref, fb2_ref, dw1_ref, o_ref):
        dv = dinv_ref[...]
        z = dv * (p0_ref[...] + p1_ref[...] + hp_ref[...]) + b3_ref[...]
        t = _lrelu(z)
        t = _lrelu(jnp.dot(t, lw1_ref[...],
                           preferred_element_type=jnp.float32) + lb1_ref[...])
        t = jnp.dot(t, lw2_ref[...],
                    preferred_element_type=jnp.float32) + lb2_ref[...]
        t = _lrelu(jnp.dot(t, fw1_ref[...],
                           preferred_element_type=jnp.float32) + fb1_ref[...])
        t = _lrelu(jnp.dot(t, fw2_ref[...],
                           preferred_element_type=jnp.float32) + fb2_ref[...])
        o_ref[...] = dv * jnp.dot(t, dw1_ref[...],
                                  preferred_element_type=jnp.float32)

    return pl.pallas_call(
        body,
        grid=(NN // BLK,),
        in_specs=[_row_spec(128), _row_spec(128), _row_spec(128), _row_spec(1),
                  _full_spec((1, 128)),
                  _full_spec((128, 32)), _full_spec((1, 32)),
                  _full_spec((32, 3)), _full_spec((1, 3)),
                  _full_spec((3, 32)), _full_spec((1, 32)),
                  _full_spec((32, 64)), _full_spec((1, 64)),
                  _full_spec((64, 128))],
        out_specs=_row_spec(128),
        out_shape=jax.ShapeDtypeStruct((NN, 128), jnp.float32),
    )(p0, p1, hp, dinv, b3.reshape(1, 128),
      lw1, lb1.reshape(1, 32), lw2, lb2.reshape(1, 3),
      fw1, fb1.reshape(1, 32), fw2, fb2.reshape(1, 64), dw1)


def _tc_final(p0, p1, hp, dinv, b):
    def body(p0_ref, p1_ref, hp_ref, dinv_ref, b_ref, o_ref):
        dv = dinv_ref[...]
        z = dv * (p0_ref[...] + p1_ref[...] + hp_ref[...]) + b_ref[...]
        o_ref[...] = jnp.tanh(_lrelu(z))

    return pl.pallas_call(
        body,
        grid=(NN // BLK,),
        in_specs=[_row_spec(128), _row_spec(128), _row_spec(128),
                  _row_spec(1), _full_spec((1, 128))],
        out_specs=_row_spec(128),
        out_shape=jax.ShapeDtypeStruct((NN, 128), jnp.float32),
    )(p0, p1, hp, dinv, b.reshape(1, 128))


def kernel(x, edge_index, enc_W1, enc_b1, enc_W2, enc_b2, enc_W3, enc_b3,
           lat_W1, lat_b1, lat_W2, lat_b2, dfc_W1, dfc_b1, dfc_W2, dfc_b2,
           dec_W1, dec_b1, dec_W2, dec_b2, dec_W3, dec_b3):
    src = edge_index[0].reshape(NCHUNK, CH)
    dst = edge_index[1].reshape(NCHUNK, CH)

    prop = _make_prop(128)

    degp = _make_deg()(dst)
    hp1, dinv = _tc_pre(degp[0], degp[1], x, _padc(enc_W1))

    agg = prop(hp1, src, dst)
    hp2 = _tc_layer(agg[0], agg[1], hp1, dinv, _padc(enc_b1), _padr(enc_W2))

    agg = prop(hp2, src, dst)
    hp3 = _tc_layer(agg[0], agg[1], hp2, dinv, enc_b2, _padc(enc_W3))

    agg = prop(hp3, src, dst)
    hp4 = _tc_mid(agg[0], agg[1], hp3, dinv, _padc(enc_b3),
                  _padr(lat_W1), lat_b1, lat_W2, lat_b2,
                  dfc_W1, dfc_b1, dfc_W2, dfc_b2, dec_W1)

    agg = prop(hp4, src, dst)
    hp5 = _tc_layer(agg[0], agg[1], hp4, dinv, dec_b1, _padc(dec_W2))

    agg = prop(hp5, src, dst)
    hp6 = _tc_layer(agg[0], agg[1], hp5, dinv, _padc(dec_b2), _padr(dec_W3))

    agg = prop(hp6, src, dst)
    return _tc_final(agg[0], agg[1], hp6, dinv, dec_b3)


# speedup vs baseline: 17.9326x; 1.0792x over previous
import functools

import jax
import jax.numpy as jnp
from jax import lax
from jax.experimental import pallas as pl
from jax.experimental.pallas import tpu as pltpu
from jax.experimental.pallas import tpu_sc as plsc

NN = 10000
EE = 320000
CH = 128
NCHUNK = EE // CH
BASE_CH = NCHUNK // 32
PER_CORE = BASE_CH * 16
LEFT0 = 2 * PER_CORE
RPT = 624
ZR = 48
BLK = 1000


def _mesh():
    return plsc.VectorSubcoreMesh(core_axis_name="c", subcore_axis_name="s")


@functools.lru_cache(maxsize=None)
def _make_prop(dim):
    @functools.partial(
        pl.kernel,
        mesh=_mesh(),
        out_type=jax.ShapeDtypeStruct((2, NN, dim), jnp.float32),
        scratch_types=[
            pltpu.VMEM((CH,), jnp.int32),
            pltpu.VMEM((CH,), jnp.int32),
            pltpu.VMEM((CH, dim), jnp.float32),
            pltpu.VMEM((CH,), jnp.int32),
            pltpu.VMEM((CH,), jnp.int32),
            pltpu.VMEM((CH, dim), jnp.float32),
            pltpu.VMEM((ZR, dim), jnp.float32),
            pltpu.VMEM_SHARED((NN, dim), jnp.float32),
            pltpu.SemaphoreType.DMA,
            pltpu.SemaphoreType.DMA,
            pltpu.SemaphoreType.DMA,
            pltpu.SemaphoreType.DMA,
        ],
        compiler_params=pltpu.CompilerParams(use_tc_tiling_on_sc=False),
    )
    def prop(h_hbm, src_hbm, dst_hbm, out_hbm,
             sa, da, ra, sb, db, rb, zb, acc, gsa, gsb, ssa, ssb):
        cid = lax.axis_index("c")
        sid = lax.axis_index("s")
        zv = jnp.zeros((16,), jnp.float32)
        for r in range(ZR):
            for k in range(dim // 16):
                zb[r, pl.ds(k * 16, 16)] = zv
        row0 = sid * RPT
        for r in range(RPT // ZR):
            pltpu.sync_copy(zb, acc.at[pl.ds(row0 + r * ZR, ZR)])

        @pl.when(sid == 0)
        def _():
            pltpu.sync_copy(zb.at[pl.ds(0, 16)], acc.at[pl.ds(16 * RPT, 16)])

        plsc.subcore_barrier()

        start = (cid * 16 + sid) * BASE_CH
        A = (sa, da, ra, gsa, ssa)
        B = (sb, db, rb, gsb, ssb)

        def idx_load(j, S):
            pltpu.sync_copy(src_hbm.at[j], S[0])
            pltpu.sync_copy(dst_hbm.at[j], S[1])

        def gather_start(S):
            pltpu.async_copy(h_hbm.at[S[0]], S[2], S[3])

        def gather_wait(S):
            pltpu.make_async_copy(h_hbm.at[S[0]], S[2], S[3]).wait()

        def scat_start(S):
            pltpu.async_copy(S[2], acc.at[S[1]], S[4], add=True)

        def scat_wait(S):
            pltpu.make_async_copy(S[2], acc.at[S[1]], S[4]).wait()

        def stage(k, X, Y, first=False):
            if not first:
                scat_wait(Y)
            idx_load(k + 1, Y)
            gather_wait(X)
            gather_start(Y)
            scat_start(X)

        idx_load(start, A)
        gather_start(A)
        stage(start, A, B, first=True)

        def body(m, carry):
            t = start + 1 + 2 * m
            stage(t, B, A)
            stage(t + 1, A, B)
            return carry

        lax.fori_loop(0, (BASE_CH - 2) // 2, body, 0)

        gather_wait(B)
        scat_wait(A)
        scat_start(B)
        scat_wait(B)

        @pl.when((cid == 0) & (sid < NCHUNK - 32 * BASE_CH))
        def _():
            idx_load(LEFT0 + sid, A)
            pltpu.async_copy(h_hbm.at[sa], ra, gsa).wait()
            pltpu.sync_copy(ra, acc.at[da], add=True)

        plsc.subcore_barrier()
        pltpu.sync_copy(acc.at[pl.ds(row0, RPT)],
                        out_hbm.at[cid, pl.ds(row0, RPT)])

        @pl.when(sid == 0)
        def _():
            pltpu.sync_copy(acc.at[pl.ds(16 * RPT, 16)],
                            out_hbm.at[cid, pl.ds(16 * RPT, 16)])

    return prop


@functools.lru_cache(maxsize=None)
def _make_deg():
    @functools.partial(
        pl.kernel,
        mesh=_mesh(),
        out_type=jax.ShapeDtypeStruct((2, NN, 16), jnp.float32),
        scratch_types=[
            pltpu.VMEM((CH,), jnp.int32),
            pltpu.VMEM((CH,), jnp.int32),
            pltpu.VMEM((CH, 16), jnp.float32),
            pltpu.VMEM((ZR, 16), jnp.float32),
            pltpu.VMEM_SHARED((NN, 16), jnp.float32),
            pltpu.SemaphoreType.DMA,
            pltpu.SemaphoreType.DMA,
        ],
        compiler_params=pltpu.CompilerParams(use_tc_tiling_on_sc=False),
    )
    def deg(dst_hbm, out_hbm, da, db, ones_v, zb, acc, ssa, ssb):
        cid = lax.axis_index("c")
        sid = lax.axis_index("s")
        zv = jnp.zeros((16,), jnp.float32)
        ov = jnp.ones((16,), jnp.float32)
        for r in range(ZR):
            zb[r, pl.ds(0, 16)] = zv
        for r in range(CH):
            ones_v[r, pl.ds(0, 16)] = ov
        row0 = sid * RPT
        for r in range(RPT // ZR):
            pltpu.sync_copy(zb, acc.at[pl.ds(row0 + r * ZR, ZR)])

        @pl.when(sid == 0)
        def _():
            pltpu.sync_copy(zb.at[pl.ds(0, 16)], acc.at[pl.ds(16 * RPT, 16)])

        plsc.subcore_barrier()

        start = (cid * 16 + sid) * BASE_CH

        def scat_start(d, sem):
            pltpu.async_copy(ones_v, acc.at[d], sem, add=True)

        def scat_wait(d, sem):
            pltpu.make_async_copy(ones_v, acc.at[d], sem).wait()

        pltpu.sync_copy(dst_hbm.at[start], da)
        scat_start(da, ssa)
        pltpu.sync_copy(dst_hbm.at[start + 1], db)
        scat_start(db, ssb)

        def body(m, carry):
            t = start + 2 + 2 * m
            scat_wait(da, ssa)
            pltpu.sync_copy(dst_hbm.at[t], da)
            scat_start(da, ssa)
            scat_wait(db, ssb)
            pltpu.sync_copy(dst_hbm.at[t + 1], db)
            scat_start(db, ssb)
            return carry

        lax.fori_loop(0, (BASE_CH - 2) // 2, body, 0)
        scat_wait(da, ssa)
        scat_wait(db, ssb)

        @pl.when((cid == 0) & (sid < NCHUNK - 32 * BASE_CH))
        def _():
            pltpu.sync_copy(dst_hbm.at[LEFT0 + sid], da)
            pltpu.sync_copy(ones_v, acc.at[da], add=True)

        plsc.subcore_barrier()
        pltpu.sync_copy(acc.at[pl.ds(row0, RPT)],
                        out_hbm.at[cid, pl.ds(row0, RPT)])

        @pl.when(sid == 0)
        def _():
            pltpu.sync_copy(acc.at[pl.ds(16 * RPT, 16)],
                            out_hbm.at[cid, pl.ds(16 * RPT, 16)])

    return deg


def _row_spec(d):
    return pl.BlockSpec((BLK, d), lambda i: (i, 0))


def _full_spec(shape):
    nd = len(shape)
    return pl.BlockSpec(shape, lambda i, _nd=nd: (0,) * _nd)


def _lrelu(v):
    return jnp.where(v >= 0, v, 0.01 * v)


def _tc_pre(d0, d1, x, w1):
    d_out = w1.shape[1]

    def body(d0_ref, d1_ref, x_ref, w_ref, hp_ref, dinv_ref):
        deg = d0_ref[:, 0:1] + d1_ref[:, 0:1] + 1.0
        dinv = 1.0 / jnp.sqrt(deg)
        h = jnp.dot(x_ref[...], w_ref[...], preferred_element_type=jnp.float32)
        hp_ref[...] = dinv * h
        dinv_ref[...] = dinv

    return pl.pallas_call(
        body,
        grid=(NN // BLK,),
        in_specs=[_row_spec(16), _row_spec(16), _row_spec(128),
                  _full_spec(w1.shape)],
        out_specs=[_row_spec(d_out), _row_spec(1)],
        out_shape=[jax.ShapeDtypeStruct((NN, d_out), jnp.float32),
                   jax.ShapeDtypeStruct((NN, 1), jnp.float32)],
    )(d0, d1, x, w1)


def _tc_layer(p0, p1, hp, dinv, b, w):
    d_in, d_out = w.shape

    def body(p0_ref, p1_ref, hp_ref, dinv_ref, b_ref, w_ref, o_ref):
        dv = dinv_ref[...]
        z = dv * (p0_ref[...] + p1_ref[...] + hp_ref[...]) + b_ref[...]
        a = _lrelu(z)
        o_ref[...] = dv * jnp.dot(a, w_ref[...],
                                  preferred_element_type=jnp.float32)

    return pl.pallas_call(
        body,
        grid=(NN // BLK,),
        in_specs=[_row_spec(d_in), _row_spec(d_in), _row_spec(d_in),
                  _row_spec(1), _full_spec((1, d_in)), _full_spec(w.shape)],
        out_specs=_row_spec(d_out),
        out_shape=jax.ShapeDtypeStruct((NN, d_out), jnp.float32),
    )(p0, p1, hp, dinv, b.reshape(1, d_in), w)


def _tc_mid(p0, p1, hp, dinv, b3, lw1, lb1, lw2, lb2, fw1, fb1, fw2, fb2, dw1):
    def body(p0_ref, p1_ref, hp_ref, dinv_ref, b3_ref,
             lw1_ref, lb1_ref, lw2_ref, lb2_ref,
             fw1_ref, fb1_ref, fw2_ref, fb2_ref, dw1_ref, o_ref):
        dv = dinv_ref[...]
        z = dv * (p0_ref[...] + p1_ref[...] + hp_ref[...]) + b3_ref[...]
        t = _lrelu(z)
        t = _lrelu(jnp.dot(t, lw1_ref[...],
                           preferred_element_type=jnp.float32) + lb1_ref[...])
        t = jnp.dot(t, lw2_ref[...],
                    preferred_element_type=jnp.float32) + lb2_ref[...]
        t = _lrelu(jnp.dot(t, fw1_ref[...],
                           preferred_element_type=jnp.float32) + fb1_ref[...])
        t = _lrelu(jnp.dot(t, fw2_ref[...],
                           preferred_element_type=jnp.float32) + fb2_ref[...])
        o_ref[...] = dv * jnp.dot(t, dw1_ref[...],
                                  preferred_element_type=jnp.float32)

    return pl.pallas_call(
        body,
        grid=(NN // BLK,),
        in_specs=[_row_spec(64), _row_spec(64), _row_spec(64), _row_spec(1),
                  _full_spec((1, 64)),
                  _full_spec((64, 32)), _full_spec((1, 32)),
                  _full_spec((32, 3)), _full_spec((1, 3)),
                  _full_spec((3, 32)), _full_spec((1, 32)),
                  _full_spec((32, 64)), _full_spec((1, 64)),
                  _full_spec((64, 128))],
        out_specs=_row_spec(128),
        out_shape=jax.ShapeDtypeStruct((NN, 128), jnp.float32),
    )(p0, p1, hp, dinv, b3.reshape(1, 64),
      lw1, lb1.reshape(1, 32), lw2, lb2.reshape(1, 3),
      fw1, fb1.reshape(1, 32), fw2, fb2.reshape(1, 64), dw1)


def _tc_final(p0, p1, hp, dinv, b):
    def body(p0_ref, p1_ref, hp_ref, dinv_ref, b_ref, o_ref):
        dv = dinv_ref[...]
        z = dv * (p0_ref[...] + p1_ref[...] + hp_ref[...]) + b_ref[...]
        o_ref[...] = jnp.tanh(_lrelu(z))

    return pl.pallas_call(
        body,
        grid=(NN // BLK,),
        in_specs=[_row_spec(128), _row_spec(128), _row_spec(128),
                  _row_spec(1), _full_spec((1, 128))],
        out_specs=_row_spec(128),
        out_shape=jax.ShapeDtypeStruct((NN, 128), jnp.float32),
    )(p0, p1, hp, dinv, b.reshape(1, 128))


def kernel(x, edge_index, enc_W1, enc_b1, enc_W2, enc_b2, enc_W3, enc_b3,
           lat_W1, lat_b1, lat_W2, lat_b2, dfc_W1, dfc_b1, dfc_W2, dfc_b2,
           dec_W1, dec_b1, dec_W2, dec_b2, dec_W3, dec_b3):
    src = edge_index[0].reshape(NCHUNK, CH)
    dst = edge_index[1].reshape(NCHUNK, CH)

    prop64 = _make_prop(64)
    prop128 = _make_prop(128)

    degp = _make_deg()(dst)
    hp1, dinv = _tc_pre(degp[0], degp[1], x, enc_W1)

    agg = prop64(hp1, src, dst)
    hp2 = _tc_layer(agg[0], agg[1], hp1, dinv, enc_b1, enc_W2)

    agg = prop128(hp2, src, dst)
    hp3 = _tc_layer(agg[0], agg[1], hp2, dinv, enc_b2, enc_W3)

    agg = prop64(hp3, src, dst)
    hp4 = _tc_mid(agg[0], agg[1], hp3, dinv, enc_b3,
                  lat_W1, lat_b1, lat_W2, lat_b2,
                  dfc_W1, dfc_b1, dfc_W2, dfc_b2, dec_W1)

    agg = prop128(hp4, src, dst)
    hp5 = _tc_layer(agg[0], agg[1], hp4, dinv, dec_b1, dec_W2)

    agg = prop64(hp5, src, dst)
    hp6 = _tc_layer(agg[0], agg[1], hp5, dinv, dec_b2, dec_W3)

    agg = prop128(hp6, src, dst)
    return _tc_final(agg[0], agg[1], hp6, dinv, dec_b3)


# speedup vs baseline: 20.6642x; 1.1523x over previous
import functools

import jax
import jax.numpy as jnp
from jax import lax
from jax.experimental import pallas as pl
from jax.experimental.pallas import tpu as pltpu
from jax.experimental.pallas import tpu_sc as plsc

NN = 10000
EE = 320000
CH = 128
NCHUNK = EE // CH
BASE_CH = NCHUNK // 32
PER_CORE = BASE_CH * 16
LEFT0 = 2 * PER_CORE
RPT = 624
ZR = 48
BLK = 1000


def _mesh():
    return plsc.VectorSubcoreMesh(core_axis_name="c", subcore_axis_name="s")


@functools.lru_cache(maxsize=None)
def _make_prop(dim):
    @functools.partial(
        pl.kernel,
        mesh=_mesh(),
        out_type=jax.ShapeDtypeStruct((2, NN, dim), jnp.float32),
        scratch_types=[
            [pltpu.VMEM((CH,), jnp.int32)] * 4,
            [pltpu.VMEM((CH,), jnp.int32)] * 4,
            [pltpu.VMEM((CH, dim), jnp.float32)] * 2,
            pltpu.VMEM((ZR, dim), jnp.float32),
            pltpu.VMEM_SHARED((NN, dim), jnp.float32),
            [pltpu.SemaphoreType.DMA] * 4,
            [pltpu.SemaphoreType.DMA] * 2,
            [pltpu.SemaphoreType.DMA] * 2,
        ],
        compiler_params=pltpu.CompilerParams(use_tc_tiling_on_sc=False),
    )
    def prop(h_hbm, src_hbm, dst_hbm, out_hbm,
             sa, da, rows, zb, acc, isem, gsem, ssem):
        cid = lax.axis_index("c")
        sid = lax.axis_index("s")
        zv = jnp.zeros((16,), jnp.float32)
        for r in range(ZR):
            for k in range(dim // 16):
                zb[r, pl.ds(k * 16, 16)] = zv
        row0 = sid * RPT
        for r in range(RPT // ZR):
            pltpu.sync_copy(zb, acc.at[pl.ds(row0 + r * ZR, ZR)])

        @pl.when(sid == 0)
        def _():
            pltpu.sync_copy(zb.at[pl.ds(0, 16)], acc.at[pl.ds(16 * RPT, 16)])

        plsc.subcore_barrier()

        start = (cid * 16 + sid) * BASE_CH
        LAST = BASE_CH - 1

        def idx_start(k, j):
            pltpu.async_copy(src_hbm.at[start + k], sa[j], isem[j])
            pltpu.async_copy(dst_hbm.at[start + k], da[j], isem[j])

        def idx_wait(k, j):
            pltpu.make_async_copy(src_hbm.at[start + k], sa[j], isem[j]).wait()
            pltpu.make_async_copy(dst_hbm.at[start + k], da[j], isem[j]).wait()

        def gather_start(j, b):
            pltpu.async_copy(h_hbm.at[sa[j]], rows[b], gsem[b])

        def gather_wait(j, b):
            pltpu.make_async_copy(h_hbm.at[sa[j]], rows[b], gsem[b]).wait()

        def scat_start(j, b):
            pltpu.async_copy(rows[b], acc.at[da[j]], ssem[b], add=True)

        def scat_wait(j, b):
            pltpu.make_async_copy(rows[b], acc.at[da[j]], ssem[b]).wait()

        def stage(k, j, b):
            gather_wait(j, b)
            scat_wait((j + 3) % 4, 1 - b)
            idx_wait(k + 1, (j + 1) % 4)
            gather_start((j + 1) % 4, 1 - b)
            scat_start(j, b)
            idx_start(k + 2, (j + 2) % 4)

        idx_start(0, 0)
        idx_wait(0, 0)
        gather_start(0, 0)
        idx_start(1, 1)
        gather_wait(0, 0)
        idx_wait(1, 1)
        gather_start(1, 1)
        scat_start(0, 0)
        idx_start(2, 2)

        def body(m, carry):
            k = 1 + 4 * m
            for i in range(4):
                stage(k + i, (1 + i) % 4, (1 + i) % 2)
            return carry

        lax.fori_loop(0, (BASE_CH - 2) // 4, body, 0)
        gather_wait(LAST % 4, 1)
        scat_wait((LAST + 3) % 4, 0)
        scat_start(LAST % 4, 1)
        scat_wait(LAST % 4, 1)
        idx_wait(LAST + 1, (LAST + 1) % 4)

        @pl.when((cid == 0) & (sid < NCHUNK - 32 * BASE_CH))
        def _():
            pltpu.sync_copy(src_hbm.at[LEFT0 + sid], sa[0])
            pltpu.sync_copy(dst_hbm.at[LEFT0 + sid], da[0])
            pltpu.async_copy(h_hbm.at[sa[0]], rows[0], gsem[0]).wait()
            pltpu.sync_copy(rows[0], acc.at[da[0]], add=True)

        plsc.subcore_barrier()
        pltpu.sync_copy(acc.at[pl.ds(row0, RPT)],
                        out_hbm.at[cid, pl.ds(row0, RPT)])

        @pl.when(sid == 0)
        def _():
            pltpu.sync_copy(acc.at[pl.ds(16 * RPT, 16)],
                            out_hbm.at[cid, pl.ds(16 * RPT, 16)])

    return prop


@functools.lru_cache(maxsize=None)
def _make_deg():
    @functools.partial(
        pl.kernel,
        mesh=_mesh(),
        out_type=jax.ShapeDtypeStruct((2, NN, 16), jnp.float32),
        scratch_types=[
            pltpu.VMEM((CH,), jnp.int32),
            pltpu.VMEM((CH,), jnp.int32),
            pltpu.VMEM((CH, 16), jnp.float32),
            pltpu.VMEM((ZR, 16), jnp.float32),
            pltpu.VMEM_SHARED((NN, 16), jnp.float32),
            pltpu.SemaphoreType.DMA,
            pltpu.SemaphoreType.DMA,
        ],
        compiler_params=pltpu.CompilerParams(use_tc_tiling_on_sc=False),
    )
    def deg(dst_hbm, out_hbm, da, db, ones_v, zb, acc, ssa, ssb):
        cid = lax.axis_index("c")
        sid = lax.axis_index("s")
        zv = jnp.zeros((16,), jnp.float32)
        ov = jnp.ones((16,), jnp.float32)
        for r in range(ZR):
            zb[r, pl.ds(0, 16)] = zv
        for r in range(CH):
            ones_v[r, pl.ds(0, 16)] = ov
        row0 = sid * RPT
        for r in range(RPT // ZR):
            pltpu.sync_copy(zb, acc.at[pl.ds(row0 + r * ZR, ZR)])

        @pl.when(sid == 0)
        def _():
            pltpu.sync_copy(zb.at[pl.ds(0, 16)], acc.at[pl.ds(16 * RPT, 16)])

        plsc.subcore_barrier()

        start = (cid * 16 + sid) * BASE_CH

        def scat_start(d, sem):
            pltpu.async_copy(ones_v, acc.at[d], sem, add=True)

        def scat_wait(d, sem):
            pltpu.make_async_copy(ones_v, acc.at[d], sem).wait()

        pltpu.sync_copy(dst_hbm.at[start], da)
        scat_start(da, ssa)
        pltpu.sync_copy(dst_hbm.at[start + 1], db)
        scat_start(db, ssb)

        def body(m, carry):
            t = start + 2 + 2 * m
            scat_wait(da, ssa)
            pltpu.sync_copy(dst_hbm.at[t], da)
            scat_start(da, ssa)
            scat_wait(db, ssb)
            pltpu.sync_copy(dst_hbm.at[t + 1], db)
            scat_start(db, ssb)
            return carry

        lax.fori_loop(0, (BASE_CH - 2) // 2, body, 0)
        scat_wait(da, ssa)
        scat_wait(db, ssb)

        @pl.when((cid == 0) & (sid < NCHUNK - 32 * BASE_CH))
        def _():
            pltpu.sync_copy(dst_hbm.at[LEFT0 + sid], da)
            pltpu.sync_copy(ones_v, acc.at[da], add=True)

        plsc.subcore_barrier()
        pltpu.sync_copy(acc.at[pl.ds(row0, RPT)],
                        out_hbm.at[cid, pl.ds(row0, RPT)])

        @pl.when(sid == 0)
        def _():
            pltpu.sync_copy(acc.at[pl.ds(16 * RPT, 16)],
                            out_hbm.at[cid, pl.ds(16 * RPT, 16)])

    return deg


def _row_spec(d):
    return pl.BlockSpec((BLK, d), lambda i: (i, 0))


def _full_spec(shape):
    nd = len(shape)
    return pl.BlockSpec(shape, lambda i, _nd=nd: (0,) * _nd)


def _lrelu(v):
    return jnp.where(v >= 0, v, 0.01 * v)


def _tc_pre(d0, d1, x, w1):
    d_out = w1.shape[1]

    def body(d0_ref, d1_ref, x_ref, w_ref, hp_ref, dinv_ref):
        deg = d0_ref[:, 0:1] + d1_ref[:, 0:1] + 1.0
        dinv = 1.0 / jnp.sqrt(deg)
        h = jnp.dot(x_ref[...], w_ref[...], preferred_element_type=jnp.float32)
        hp_ref[...] = dinv * h
        dinv_ref[...] = dinv

    return pl.pallas_call(
        body,
        grid=(NN // BLK,),
        in_specs=[_row_spec(16), _row_spec(16), _row_spec(128),
                  _full_spec(w1.shape)],
        out_specs=[_row_spec(d_out), _row_spec(1)],
        out_shape=[jax.ShapeDtypeStruct((NN, d_out), jnp.float32),
                   jax.ShapeDtypeStruct((NN, 1), jnp.float32)],
    )(d0, d1, x, w1)


def _tc_layer(p0, p1, hp, dinv, b, w):
    d_in, d_out = w.shape

    def body(p0_ref, p1_ref, hp_ref, dinv_ref, b_ref, w_ref, o_ref):
        dv = dinv_ref[...]
        z = dv * (p0_ref[...] + p1_ref[...] + hp_ref[...]) + b_ref[...]
        a = _lrelu(z)
        o_ref[...] = dv * jnp.dot(a, w_ref[...],
                                  preferred_element_type=jnp.float32)

    return pl.pallas_call(
        body,
        grid=(NN // BLK,),
        in_specs=[_row_spec(d_in), _row_spec(d_in), _row_spec(d_in),
                  _row_spec(1), _full_spec((1, d_in)), _full_spec(w.shape)],
        out_specs=_row_spec(d_out),
        out_shape=jax.ShapeDtypeStruct((NN, d_out), jnp.float32),
    )(p0, p1, hp, dinv, b.reshape(1, d_in), w)


def _tc_mid(p0, p1, hp, dinv, b3, lw1, lb1, lw2, lb2, fw1, fb1, fw2, fb2, dw1):
    def body(p0_ref, p1_ref, hp_ref, dinv_ref, b3_ref,
             lw1_ref, lb1_ref, lw2_ref, lb2_ref,
             fw1_ref, fb1_ref, fw2_ref, fb2_ref, dw1_ref, o_ref):
        dv = dinv_ref[...]
        z = dv * (p0_ref[...] + p1_ref[...] + hp_ref[...]) + b3_ref[...]
        t = _lrelu(z)
        t = _lrelu(jnp.dot(t, lw1_ref[...],
                           preferred_element_type=jnp.float32) + lb1_ref[...])
        t = jnp.dot(t, lw2_ref[...],
                    preferred_element_type=jnp.float32) + lb2_ref[...]
        t = _lrelu(jnp.dot(t, fw1_ref[...],
                           preferred_element_type=jnp.float32) + fb1_ref[...])
        t = _lrelu(jnp.dot(t, fw2_ref[...],
                           preferred_element_type=jnp.float32) + fb2_ref[...])
        o_ref[...] = dv * jnp.dot(t, dw1_ref[...],
                                  preferred_element_type=jnp.float32)

    return pl.pallas_call(
        body,
        grid=(NN // BLK,),
        in_specs=[_row_spec(64), _row_spec(64), _row_spec(64), _row_spec(1),
                  _full_spec((1, 64)),
                  _full_spec((64, 32)), _full_spec((1, 32)),
                  _full_spec((32, 3)), _full_spec((1, 3)),
                  _full_spec((3, 32)), _full_spec((1, 32)),
                  _full_spec((32, 64)), _full_spec((1, 64)),
                  _full_spec((64, 128))],
        out_specs=_row_spec(128),
        out_shape=jax.ShapeDtypeStruct((NN, 128), jnp.float32),
    )(p0, p1, hp, dinv, b3.reshape(1, 64),
      lw1, lb1.reshape(1, 32), lw2, lb2.reshape(1, 3),
      fw1, fb1.reshape(1, 32), fw2, fb2.reshape(1, 64), dw1)


def _tc_final(p0, p1, hp, dinv, b):
    def body(p0_ref, p1_ref, hp_ref, dinv_ref, b_ref, o_ref):
        dv = dinv_ref[...]
        z = dv * (p0_ref[...] + p1_ref[...] + hp_ref[...]) + b_ref[...]
        o_ref[...] = jnp.tanh(_lrelu(z))

    return pl.pallas_call(
        body,
        grid=(NN // BLK,),
        in_specs=[_row_spec(128), _row_spec(128), _row_spec(128),
                  _row_spec(1), _full_spec((1, 128))],
        out_specs=_row_spec(128),
        out_shape=jax.ShapeDtypeStruct((NN, 128), jnp.float32),
    )(p0, p1, hp, dinv, b.reshape(1, 128))


def kernel(x, edge_index, enc_W1, enc_b1, enc_W2, enc_b2, enc_W3, enc_b3,
           lat_W1, lat_b1, lat_W2, lat_b2, dfc_W1, dfc_b1, dfc_W2, dfc_b2,
           dec_W1, dec_b1, dec_W2, dec_b2, dec_W3, dec_b3):
    src = edge_index[0].reshape(NCHUNK, CH)
    dst = edge_index[1].reshape(NCHUNK, CH)

    prop64 = _make_prop(64)
    prop128 = _make_prop(128)

    degp = _make_deg()(dst)
    hp1, dinv = _tc_pre(degp[0], degp[1], x, enc_W1)

    agg = prop64(hp1, src, dst)
    hp2 = _tc_layer(agg[0], agg[1], hp1, dinv, enc_b1, enc_W2)

    agg = prop128(hp2, src, dst)
    hp3 = _tc_layer(agg[0], agg[1], hp2, dinv, enc_b2, enc_W3)

    agg = prop64(hp3, src, dst)
    hp4 = _tc_mid(agg[0], agg[1], hp3, dinv, enc_b3,
                  lat_W1, lat_b1, lat_W2, lat_b2,
                  dfc_W1, dfc_b1, dfc_W2, dfc_b2, dec_W1)

    agg = prop128(hp4, src, dst)
    hp5 = _tc_layer(agg[0], agg[1], hp4, dinv, dec_b1, dec_W2)

    agg = prop64(hp5, src, dst)
    hp6 = _tc_layer(agg[0], agg[1], hp5, dinv, dec_b2, dec_W3)

    agg = prop128(hp6, src, dst)
    return _tc_final(agg[0], agg[1], hp6, dinv, dec_b3)
